# Initial kernel scaffold; baseline (speedup 1.0000x reference)
import jax
import jax.numpy as jnp
from jax.experimental import pallas as pl


def kernel(h, edge_index, e, W0, b0, g0, bt0, W1, b1, g1, bt1, W2, b2, g2, bt2, W3, b3, g3, bt3, Wr, br):
    raise NotImplementedError("write your pallas kernel here")


# speedup vs baseline: 8.4611x; 8.4611x over previous
import functools

import jax
import jax.numpy as jnp
from jax import lax
from jax.experimental import pallas as pl
from jax.experimental.pallas import tpu as pltpu
from jax.experimental.pallas import tpu_sc as plsc

N = 10000
D = 128
EPS = 1e-5

NC = 2
NS = 16
K = 64
CP = 80
NPH = 2
C = CP * NPH
E = 320000
EPAD = NC * NS * C * K
NPAD = 10112
R = NPAD // NS

_mesh = plsc.VectorSubcoreMesh(core_axis_name="c", subcore_axis_name="s",
                               num_cores=NC, num_subcores=NS)


@functools.partial(
    pl.kernel,
    out_type=jax.ShapeDtypeStruct((NC, NPAD, D), jnp.float32),
    mesh=_mesh,
    scratch_types=[
        pltpu.VMEM((CP + 1, K), jnp.int32),
        pltpu.VMEM((CP + 1, K), jnp.int32),
        pltpu.VMEM((K, D), jnp.float32),
        pltpu.VMEM((K, D), jnp.float32),
        pltpu.SemaphoreType.DMA,
        pltpu.SemaphoreType.DMA,
        pltpu.VMEM_SHARED((NPAD, D), jnp.float32),
    ],
)
def _sc_agg(x_hbm, src_hbm, dst_hbm, z128_hbm, out_hbm,
            src_v, dst_v, buf0, buf1, sem0, sem1, agg_sh):
    c = lax.axis_index("c")
    s = lax.axis_index("s")
    pltpu.sync_copy(z128_hbm.at[pl.ds(s * R, R)], agg_sh.at[pl.ds(s * R, R)])
    plsc.subcore_barrier()

    for ph in range(NPH):
        pltpu.sync_copy(src_hbm.at[c, s, ph], src_v)
        pltpu.sync_copy(dst_hbm.at[c, s, ph], dst_v)
        pltpu.async_copy(x_hbm.at[src_v.at[0]], buf0, sem0)

        def pair(j2, carry):
            a = 2 * j2
            pltpu.async_copy(x_hbm.at[src_v.at[a + 1]], buf1, sem1)
            pltpu.make_async_copy(x_hbm.at[src_v.at[a]], buf0, sem0).wait()
            pltpu.sync_copy(buf0, agg_sh.at[dst_v.at[a]], add=True)
            pltpu.async_copy(x_hbm.at[src_v.at[a + 2]], buf0, sem0)
            pltpu.make_async_copy(x_hbm.at[src_v.at[a + 1]], buf1, sem1).wait()
            pltpu.sync_copy(buf1, agg_sh.at[dst_v.at[a + 1]], add=True)
            return carry

        lax.fori_loop(0, CP // 2, pair, 0)
        pltpu.make_async_copy(x_hbm.at[src_v.at[CP]], buf0, sem0).wait()

    plsc.subcore_barrier()
    pltpu.sync_copy(agg_sh.at[pl.ds(s * R, R)], out_hbm.at[c, pl.ds(s * R, R)])


@functools.partial(
    pl.kernel,
    out_type=jax.ShapeDtypeStruct((NC, NPAD, D), jnp.float32),
    mesh=_mesh,
    scratch_types=[
        pltpu.VMEM((CP + 1, K), jnp.int32),
        pltpu.VMEM((K, D), jnp.float32),
        pltpu.VMEM_SHARED((NPAD, D), jnp.float32),
    ],
)
def _sc_deg(dst_hbm, z128_hbm, ones_hbm, deg_hbm, dst_v, ones_v, deg_sh):
    c = lax.axis_index("c")
    s = lax.axis_index("s")
    pltpu.sync_copy(z128_hbm.at[pl.ds(s * R, R)], deg_sh.at[pl.ds(s * R, R)])
    pltpu.sync_copy(ones_hbm, ones_v)
    plsc.subcore_barrier()

    def chunk(j, carry):
        pltpu.sync_copy(ones_v, deg_sh.at[dst_v.at[j]], add=True)
        return carry

    for ph in range(NPH):
        pltpu.sync_copy(dst_hbm.at[c, s, ph], dst_v)
        lax.fori_loop(0, CP, chunk, 0)

    plsc.subcore_barrier()
    pltpu.sync_copy(deg_sh.at[pl.ds(s * R, R)], deg_hbm.at[c, pl.ds(s * R, R)])


def _dot(a, b):
    return jnp.dot(a, b, preferred_element_type=jnp.float32,
                   precision=lax.Precision.HIGHEST)


def _bn_relu(z, g, bt):
    mean = jnp.mean(z, axis=0, keepdims=True)
    var = jnp.mean((z - mean) ** 2, axis=0, keepdims=True)
    zn = (z - mean) * lax.rsqrt(var + EPS) * g + bt
    return jnp.maximum(zn, 0.0)


def _tc_invd_body(dp_ref, invd_ref):
    deg = dp_ref[0, :N, 0:1] + dp_ref[1, :N, 0:1]
    invd_ref[...] = 1.0 / jnp.maximum(deg, 1.0)


_tc_invd = pl.pallas_call(
    _tc_invd_body,
    out_shape=jax.ShapeDtypeStruct((N, 1), jnp.float32),
)

BLK = 2000


def _tc_z_body(x_ref, p_ref, invd_ref, ws_ref, wn_ref, b_ref, z_ref):
    agg = (p_ref[0] + p_ref[1]) * invd_ref[...]
    z_ref[...] = (_dot(x_ref[...], ws_ref[...]) + _dot(agg, wn_ref[...])
                  + b_ref[...])


_tc_z = pl.pallas_call(
    _tc_z_body,
    grid=(N // BLK,),
    in_specs=[
        pl.BlockSpec((BLK, D), lambda i: (i, 0)),
        pl.BlockSpec((NC, BLK, D), lambda i: (0, i, 0)),
        pl.BlockSpec((BLK, 1), lambda i: (i, 0)),
        pl.BlockSpec((D, D), lambda i: (0, 0)),
        pl.BlockSpec((D, D), lambda i: (0, 0)),
        pl.BlockSpec((1, D), lambda i: (0, 0)),
    ],
    out_specs=pl.BlockSpec((BLK, D), lambda i: (i, 0)),
    out_shape=jax.ShapeDtypeStruct((N, D), jnp.float32),
)


def _tc_bn_body(z_ref, g_ref, bt_ref, o_ref):
    o_ref[...] = _bn_relu(z_ref[...], g_ref[...], bt_ref[...])


_tc_bn = pl.pallas_call(
    _tc_bn_body,
    out_shape=jax.ShapeDtypeStruct((N, D), jnp.float32),
)


def _tc_bn_read_body(z_ref, g_ref, bt_ref, wr_ref, br_ref, o_ref):
    x4 = _bn_relu(z_ref[...], g_ref[...], bt_ref[...])
    o_ref[...] = _dot(x4, wr_ref[...]) + br_ref[...]


_tc_bn_read = pl.pallas_call(
    _tc_bn_read_body,
    out_shape=jax.ShapeDtypeStruct((N, D), jnp.float32),
)


def kernel(h, edge_index, e, W0, b0, g0, bt0, W1, b1, g1, bt1,
           W2, b2, g2, bt2, W3, b3, g3, bt3, Wr, br):
    src = edge_index[0].astype(jnp.int32)
    dst = edge_index[1].astype(jnp.int32)
    pad = EPAD - E
    pad_src = jnp.arange(pad, dtype=jnp.int32) % N
    pad_dst = N + (jnp.arange(pad, dtype=jnp.int32) % (NPAD - N))
    srcp = jnp.concatenate([src, pad_src]).reshape(NC, NS, NPH, CP, K)
    dstp = jnp.concatenate([dst, pad_dst]).reshape(NC, NS, NPH, CP, K)
    sent = jnp.arange(NC * NS * NPH * K, dtype=jnp.int32
                      ).reshape(NC, NS, NPH, 1, K)
    src4 = jnp.concatenate([srcp, sent % N], axis=3)
    dst4 = jnp.concatenate([dstp, N + sent % (NPAD - N)], axis=3)
    z128 = jnp.zeros((NPAD, D), jnp.float32)
    ones128 = jnp.ones((K, D), jnp.float32)

    layers = [(W0, b0, g0, bt0), (W1, b1, g1, bt1),
              (W2, b2, g2, bt2), (W3, b3, g3, bt3)]
    split = [(W[:D], W[D:], b.reshape(1, D), g.reshape(1, D),
              bt.reshape(1, D)) for (W, b, g, bt) in layers]

    dp = _sc_deg(dst4, z128, ones128)
    invd = _tc_invd(dp)

    x = h
    for i in range(3):
        p = _sc_agg(x, src4, dst4, z128)
        ws, wn, b, g, bt = split[i]
        z = _tc_z(x, p, invd, ws, wn, b)
        x = _tc_bn(z, g, bt)

    p = _sc_agg(x, src4, dst4, z128)
    ws, wn, b, g, bt = split[3]
    z = _tc_z(x, p, invd, ws, wn, b)
    return _tc_bn_read(z, g, bt, Wr, br.reshape(1, D))
